# baseline (speedup 1.0000x reference)
import functools

import jax
import jax.numpy as jnp
from jax import lax
from jax.experimental import pallas as pl
from jax.experimental.pallas import tpu as pltpu
from jax.experimental.pallas import tpu_sc as plsc

N_ATOMS = 131072
D_FEAT = 512
D_OUT = 64
N_STRUCT = 1024

N_SC = 32768
N_TC = N_ATOMS - N_SC

BLOCK_ATOMS = 8192
SEG_SEL = 64
SEG_STORE = SEG_SEL + 8

NW = 32
S_PER_W = N_STRUCT // NW
K_ROWS = 32


def _scatter_window(out_ref, yb16, ids, min_id, k):
    win_lo = min_id + k * SEG_SEL
    base = (jnp.minimum(win_lo, N_STRUCT - SEG_STORE) // 8) * 8
    rel = ids - base
    sel = (ids >= win_lo) & (ids < win_lo + SEG_SEL)
    rows = jax.lax.broadcasted_iota(jnp.int32, (SEG_STORE, BLOCK_ATOMS), 0)
    oh = ((rows == rel[None, :]) & sel[None, :]).astype(jnp.bfloat16)
    part = jnp.dot(oh, yb16, preferred_element_type=jnp.float32)
    out_ref[pl.ds(base, SEG_STORE), :] += part


def _fused_kernel(ids_ref, x_ref, w_ref, b_ref, out_ref):
    i = pl.program_id(0)

    @pl.when(i == 0)
    def _init():
        out_ref[...] = jnp.zeros_like(out_ref)

    xb16 = x_ref[...].astype(jnp.bfloat16)
    yb = jnp.dot(xb16, w_ref[...], preferred_element_type=jnp.float32)
    yb16 = (yb + b_ref[...]).astype(jnp.bfloat16)

    ids = ids_ref[0, 0, :]
    min_id = jnp.min(ids)
    max_id = jnp.max(ids)
    nwin = (max_id - min_id) // SEG_SEL + 1

    _scatter_window(out_ref, yb16, ids, min_id, 0)

    @pl.when(nwin > 1)
    def _rest():
        jax.lax.fori_loop(
            1, nwin,
            lambda k, c: (_scatter_window(out_ref, yb16, ids, min_id, k), c)[1],
            0)


def _sc_kernel(x_hbm, ids_hbm, stag_hbm, cnt_hbm, ids_v, acc_v, cnt_v, buf_v):
    core = lax.axis_index("c")
    sub = lax.axis_index("s")
    w = sub * 2 + core
    s_lo = (w * S_PER_W).astype(jnp.int32)

    pltpu.sync_copy(ids_hbm, ids_v.at[pl.ds(0, N_SC)])

    def _id_at(g):
        return ids_v[pl.ds(g, 16)][0]

    zero16 = jnp.zeros((16,), jnp.float32)
    nq = D_FEAT // 16

    def _zacc(i, c):
        r = i // nq
        q = i - r * nq
        acc_v[r, pl.ds(q * 16, 16)] = zero16
        return c

    lax.fori_loop(0, S_PER_W * nq, _zacc, 0)

    def _zcnt(i, c):
        cnt_v[i, :] = zero16
        return c

    lax.fori_loop(0, S_PER_W, _zcnt, 0)

    def _lower_bound(target):
        pos = jnp.int32(0)
        bit = N_SC
        while bit >= 1:
            npos = pos + bit
            ok = (npos <= N_SC) & (_id_at(npos - 1) < target)
            pos = jnp.where(ok, npos, pos)
            bit //= 2
        return pos

    a_lo = _lower_bound(s_lo)
    a_hi = _lower_bound(s_lo + S_PER_W)

    one16 = jnp.ones((16,), jnp.float32)

    def chunk_body(c):
        cs = a_lo + c * K_ROWS
        raw = jnp.minimum(cs, N_SC - K_ROWS - 8)
        dma_start = pl.multiple_of((raw // 8) * 8, 8)
        off = cs - dma_start
        pltpu.sync_copy(x_hbm.at[pl.ds(N_TC + dma_start, K_ROWS + 8)], buf_v)
        rows = jnp.minimum(K_ROWS, a_hi - cs)

        @pl.loop(0, rows)
        def _rows(r):
            g = cs + r
            rel = _id_at(g) - s_lo
            br = off + r
            for q in range(nq):
                sl = pl.ds(q * 16, 16)
                plsc.addupdate(acc_v.at[rel, sl], buf_v[br, sl])
            plsc.addupdate(cnt_v.at[rel], one16)

    nch = (a_hi - a_lo + K_ROWS - 1) // K_ROWS
    pl.loop(0, nch)(chunk_body)

    s_lo_a = pl.multiple_of(s_lo, 8)
    pltpu.sync_copy(acc_v, stag_hbm.at[pl.ds(s_lo_a, S_PER_W)])
    pltpu.sync_copy(cnt_v, cnt_hbm.at[pl.ds(s_lo_a, S_PER_W)])


def _combine_kernel(part_ref, stag_ref, cnt_ref, w_ref, b_ref, out_ref):
    y = jax.lax.dot_general(
        stag_ref[...], w_ref[...], (((1,), (0,)), ((), ())),
        precision=jax.lax.Precision.HIGHEST,
        preferred_element_type=jnp.float32)
    cnt = cnt_ref[:, 0:1]
    out_ref[...] = part_ref[...] + y + cnt * b_ref[...]


@jax.jit
def kernel(x, structural_indices, W, b):
    ids32 = structural_indices.astype(jnp.int32)
    w16 = W.astype(jnp.bfloat16)
    b2 = b.reshape(1, D_OUT)

    nb = N_TC // BLOCK_ATOMS
    ids3 = ids32[:N_TC].reshape(nb, 1, BLOCK_ATOMS)
    partial = pl.pallas_call(
        _fused_kernel,
        grid=(nb,),
        in_specs=[
            pl.BlockSpec((1, 1, BLOCK_ATOMS), lambda i: (i, 0, 0)),
            pl.BlockSpec((BLOCK_ATOMS, D_FEAT), lambda i: (i, 0)),
            pl.BlockSpec((D_FEAT, D_OUT), lambda i: (0, 0)),
            pl.BlockSpec((1, D_OUT), lambda i: (0, 0)),
        ],
        out_specs=pl.BlockSpec((N_STRUCT, D_OUT), lambda i: (0, 0)),
        out_shape=jax.ShapeDtypeStruct((N_STRUCT, D_OUT), jnp.float32),
    )(ids3, x, w16, b2)

    ids_tail = ids32[N_TC:]
    sck = pl.kernel(
        _sc_kernel,
        mesh=plsc.VectorSubcoreMesh(core_axis_name="c", subcore_axis_name="s"),
        out_type=[
            jax.ShapeDtypeStruct((N_STRUCT, D_FEAT), jnp.float32),
            jax.ShapeDtypeStruct((N_STRUCT, 16), jnp.float32),
        ],
        scratch_types=[
            pltpu.VMEM((N_SC + 16,), jnp.int32),
            pltpu.VMEM((S_PER_W, D_FEAT), jnp.float32),
            pltpu.VMEM((S_PER_W, 16), jnp.float32),
            pltpu.VMEM((K_ROWS + 8, D_FEAT), jnp.float32),
        ],
    )
    staging, cnt = sck(x, ids_tail)

    out = pl.pallas_call(
        _combine_kernel,
        in_specs=[
            pl.BlockSpec((N_STRUCT, D_OUT), lambda: (0, 0)),
            pl.BlockSpec((N_STRUCT, D_FEAT), lambda: (0, 0)),
            pl.BlockSpec((N_STRUCT, 16), lambda: (0, 0)),
            pl.BlockSpec((D_FEAT, D_OUT), lambda: (0, 0)),
            pl.BlockSpec((1, D_OUT), lambda: (0, 0)),
        ],
        out_specs=pl.BlockSpec((N_STRUCT, D_OUT), lambda: (0, 0)),
        out_shape=jax.ShapeDtypeStruct((N_STRUCT, D_OUT), jnp.float32),
    )(partial, staging, cnt, W, b2)
    return out


# speedup vs baseline: 7.9215x; 7.9215x over previous
import jax
import jax.numpy as jnp
from jax.experimental import pallas as pl

N_ATOMS = 131072
D_FEAT = 512
D_OUT = 64
N_STRUCT = 1024

BLOCK_ATOMS = 8192
SEG_SEL = 64
SEG_STORE = SEG_SEL + 8


def _scatter_window(out_ref, yb16, ids, min_id, k):
    win_lo = min_id + k * SEG_SEL
    base = (jnp.minimum(win_lo, N_STRUCT - SEG_STORE) // 8) * 8
    rel = ids - base
    sel = (ids >= win_lo) & (ids < win_lo + SEG_SEL)
    rows = jax.lax.broadcasted_iota(jnp.int32, (SEG_STORE, BLOCK_ATOMS), 0)
    oh = ((rows == rel[None, :]) & sel[None, :]).astype(jnp.bfloat16)
    part = jnp.dot(oh, yb16, preferred_element_type=jnp.float32)
    out_ref[pl.ds(base, SEG_STORE), :] += part


def _fused_kernel(ids_ref, x_ref, w_ref, b_ref, out_ref):
    i = pl.program_id(0)

    @pl.when(i == 0)
    def _init():
        out_ref[...] = jnp.zeros_like(out_ref)

    xb16 = x_ref[...].astype(jnp.bfloat16)
    yb = jnp.dot(xb16, w_ref[...], preferred_element_type=jnp.float32)
    yb16 = (yb + b_ref[...]).astype(jnp.bfloat16)

    ids = ids_ref[0, 0, :]
    min_id = jnp.min(ids)
    max_id = jnp.max(ids)
    nwin = (max_id - min_id) // SEG_SEL + 1

    _scatter_window(out_ref, yb16, ids, min_id, 0)

    @pl.when(nwin > 1)
    def _rest():
        jax.lax.fori_loop(
            1, nwin,
            lambda k, c: (_scatter_window(out_ref, yb16, ids, min_id, k), c)[1],
            0)


@jax.jit
def kernel(x, structural_indices, W, b):
    nb = N_ATOMS // BLOCK_ATOMS
    ids3 = structural_indices.astype(jnp.int32).reshape(nb, 1, BLOCK_ATOMS)
    w16 = W.astype(jnp.bfloat16)
    b2 = b.reshape(1, D_OUT)
    out = pl.pallas_call(
        _fused_kernel,
        grid=(nb,),
        in_specs=[
            pl.BlockSpec((1, 1, BLOCK_ATOMS), lambda i: (i, 0, 0)),
            pl.BlockSpec((BLOCK_ATOMS, D_FEAT), lambda i: (i, 0)),
            pl.BlockSpec((D_FEAT, D_OUT), lambda i: (0, 0)),
            pl.BlockSpec((1, D_OUT), lambda i: (0, 0)),
        ],
        out_specs=pl.BlockSpec((N_STRUCT, D_OUT), lambda i: (0, 0)),
        out_shape=jax.ShapeDtypeStruct((N_STRUCT, D_OUT), jnp.float32),
    )(ids3, x, w16, b2)
    return out


# speedup vs baseline: 8.1403x; 1.0276x over previous
import jax
import jax.numpy as jnp
from jax.experimental import pallas as pl

N_ATOMS = 131072
D_FEAT = 512
D_OUT = 64
N_STRUCT = 1024

BLOCK_ATOMS = 8192
SEG_SEL = 32
SEG_STORE = SEG_SEL + 8


def _scatter_window(out_ref, yb16, ids, min_id, k):
    win_lo = min_id + k * SEG_SEL
    base = (jnp.minimum(win_lo, N_STRUCT - SEG_STORE) // 8) * 8
    rel = ids - base
    sel = (ids >= win_lo) & (ids < win_lo + SEG_SEL)
    rows = jax.lax.broadcasted_iota(jnp.int32, (SEG_STORE, BLOCK_ATOMS), 0)
    oh = ((rows == rel[None, :]) & sel[None, :]).astype(jnp.bfloat16)
    part = jnp.dot(oh, yb16, preferred_element_type=jnp.float32)
    out_ref[pl.ds(base, SEG_STORE), :] += part


def _fused_kernel(ids_ref, x_ref, w_ref, b_ref, out_ref):
    i = pl.program_id(0)

    @pl.when(i == 0)
    def _init():
        out_ref[...] = jnp.zeros_like(out_ref)

    xb16 = x_ref[...].astype(jnp.bfloat16)
    yb = jnp.dot(xb16, w_ref[...], preferred_element_type=jnp.float32)
    yb16 = (yb + b_ref[...]).astype(jnp.bfloat16)

    ids = ids_ref[0, 0, :]
    min_id = jnp.min(ids)
    max_id = jnp.max(ids)
    nwin = (max_id - min_id) // SEG_SEL + 1

    _scatter_window(out_ref, yb16, ids, min_id, 0)

    @pl.when(nwin > 1)
    def _rest():
        jax.lax.fori_loop(
            1, nwin,
            lambda k, c: (_scatter_window(out_ref, yb16, ids, min_id, k), c)[1],
            0)


@jax.jit
def kernel(x, structural_indices, W, b):
    nb = N_ATOMS // BLOCK_ATOMS
    ids3 = structural_indices.astype(jnp.int32).reshape(nb, 1, BLOCK_ATOMS)
    w16 = W.astype(jnp.bfloat16)
    b2 = b.reshape(1, D_OUT)
    out = pl.pallas_call(
        _fused_kernel,
        grid=(nb,),
        in_specs=[
            pl.BlockSpec((1, 1, BLOCK_ATOMS), lambda i: (i, 0, 0)),
            pl.BlockSpec((BLOCK_ATOMS, D_FEAT), lambda i: (i, 0)),
            pl.BlockSpec((D_FEAT, D_OUT), lambda i: (0, 0)),
            pl.BlockSpec((1, D_OUT), lambda i: (0, 0)),
        ],
        out_specs=pl.BlockSpec((N_STRUCT, D_OUT), lambda i: (0, 0)),
        out_shape=jax.ShapeDtypeStruct((N_STRUCT, D_OUT), jnp.float32),
    )(ids3, x, w16, b2)
    return out
